# baseline (speedup 1.0000x reference)
import functools

import jax
import jax.numpy as jnp
from jax import lax
from jax.experimental import pallas as pl
from jax.experimental.pallas import tpu as pltpu
from jax.experimental.pallas import tpu_sc as plsc

NC = 2
NS = 16
NW = NC * NS
L = 16

CHUNK = 80
GROUPS = CHUNK // L
FLUSH = 25
STRIPE = 1000
SH_TILES = 7


def _tec_body(D, per_w, n_nodes, table_hbm, src_hbm, tgt_hbm, out_hbm,
              table_sh, sidx0, tidx0, sidx1, tidx1,
              srows0, trows0, srows1, trows1,
              acc_v, out_v, sem0, sem1, isem0, isem1):
  wid = lax.axis_index("s") * NC + lax.axis_index("c")
  sid = lax.axis_index("s")
  use_sh = sid < SH_TILES
  n_chunks = per_w // CHUNK
  base = wid * per_w
  nslice = D // L
  bufs = ((sidx0, tidx0, srows0, trows0, sem0, isem0),
          (sidx1, tidx1, srows1, trows1, sem1, isem1))

  @pl.when(sid < n_nodes // STRIPE)
  def _():
    off = pl.multiple_of(sid * STRIPE, 8)
    pltpu.sync_copy(table_hbm.at[pl.ds(off, STRIPE)], table_sh.at[pl.ds(off, STRIPE)])

  plsc.subcore_barrier()

  def idx_refs(ci):
    off = pl.multiple_of(base + ci * CHUNK, 8)
    return src_hbm.at[pl.ds(off, CHUNK)], tgt_hbm.at[pl.ds(off, CHUNK)]

  def fire_idx(ci, b):
    sidx, tidx, _, _, _, isem = bufs[b]
    shbm, thbm = idx_refs(ci)
    pltpu.async_copy(shbm, sidx, isem)
    pltpu.async_copy(thbm, tidx, isem)

  def wait_idx(ci, b):
    sidx, tidx, _, _, _, isem = bufs[b]
    shbm, thbm = idx_refs(ci)
    pltpu.make_async_copy(shbm, sidx, isem).wait()
    pltpu.make_async_copy(thbm, tidx, isem).wait()

  def fire_gather(b):
    sidx, tidx, srows, trows, sem, _ = bufs[b]

    @pl.when(use_sh)
    def _():
      pltpu.async_copy(table_sh.at[sidx], srows, sem)
      pltpu.async_copy(table_sh.at[tidx], trows, sem)

    @pl.when(jnp.logical_not(use_sh))
    def _():
      pltpu.async_copy(table_hbm.at[sidx], srows, sem)
      pltpu.async_copy(table_hbm.at[tidx], trows, sem)

  def wait_gather(b):
    sidx, tidx, srows, trows, sem, _ = bufs[b]

    @pl.when(use_sh)
    def _():
      pltpu.make_async_copy(table_sh.at[sidx], srows, sem).wait()
      pltpu.make_async_copy(table_sh.at[tidx], trows, sem).wait()

    @pl.when(jnp.logical_not(use_sh))
    def _():
      pltpu.make_async_copy(table_hbm.at[sidx], srows, sem).wait()
      pltpu.make_async_copy(table_hbm.at[tidx], trows, sem).wait()

  def compute(ci, b):
    _, _, srows, trows, _, _ = bufs[b]
    slot = lax.rem(ci, FLUSH)

    def group_body(g, c2):
      eb = g * L
      for j in range(L):
        e = eb + j
        acc = srows[e, pl.ds(0, L)] * trows[e, pl.ds(0, L)]
        for k in range(1, nslice):
          acc = acc + srows[e, pl.ds(k * L, L)] * trows[e, pl.ds(k * L, L)]
        acc_v[pl.ds(j * L, L)] = acc
      rows = lax.iota(jnp.int32, L) * L
      tot = plsc.load_gather(acc_v, [rows])
      for i in range(1, L):
        tot = tot + plsc.load_gather(acc_v, [rows + i])
      out_v[pl.ds(slot * CHUNK + eb, L)] = 1.0 / (1.0 + jnp.exp(-tot))
      return c2

    lax.fori_loop(0, GROUPS, group_body, 0)

  s0hbm, t0hbm = idx_refs(0)
  pltpu.sync_copy(s0hbm, sidx0)
  pltpu.sync_copy(t0hbm, tidx0)
  fire_gather(0)
  fire_idx(1, 1)

  def outer(io, carry):
    for b in range(2):
      ci = io * 2 + b
      ob = 1 - b

      @pl.when(ci + 1 < n_chunks)
      def _():
        wait_idx(ci + 1, ob)
        fire_gather(ob)

      wait_gather(b)
      compute(ci, b)

      @pl.when(ci + 2 < n_chunks)
      def _():
        fire_idx(ci + 2, b)

      @pl.when(lax.rem(ci, FLUSH) == FLUSH - 1)
      def _():
        foff = pl.multiple_of(base + (ci - (FLUSH - 1)) * CHUNK, 8)
        pltpu.sync_copy(out_v, out_hbm.at[pl.ds(foff, FLUSH * CHUNK)])

    return carry

  lax.fori_loop(0, n_chunks // 2, outer, 0)


def _link_predict(table, src, tgt):
  E = src.shape[0]
  n_nodes, D = table.shape
  assert E % NW == 0
  per_w = E // NW
  n_chunks = per_w // CHUNK
  assert per_w % CHUNK == 0 and D % L == 0
  assert n_chunks % 2 == 0 and n_chunks % FLUSH == 0
  assert n_nodes % STRIPE == 0 and n_nodes // STRIPE <= NS

  mesh = plsc.VectorSubcoreMesh(core_axis_name="c", subcore_axis_name="s")
  k = pl.kernel(
      functools.partial(_tec_body, D, per_w, n_nodes),
      out_type=jax.ShapeDtypeStruct((E,), jnp.float32),
      mesh=mesh,
      compiler_params=pltpu.CompilerParams(needs_layout_passes=False),
      scratch_types=[
          pltpu.VMEM_SHARED((n_nodes, D), jnp.float32),
          pltpu.VMEM((CHUNK,), jnp.int32),
          pltpu.VMEM((CHUNK,), jnp.int32),
          pltpu.VMEM((CHUNK,), jnp.int32),
          pltpu.VMEM((CHUNK,), jnp.int32),
          pltpu.VMEM((CHUNK, D), jnp.float32),
          pltpu.VMEM((CHUNK, D), jnp.float32),
          pltpu.VMEM((CHUNK, D), jnp.float32),
          pltpu.VMEM((CHUNK, D), jnp.float32),
          pltpu.VMEM((L * L,), jnp.float32),
          pltpu.VMEM((FLUSH * CHUNK,), jnp.float32),
          pltpu.SemaphoreType.DMA,
          pltpu.SemaphoreType.DMA,
          pltpu.SemaphoreType.DMA,
          pltpu.SemaphoreType.DMA,
      ],
  )
  return k(table, src, tgt)


def kernel(node_embedding_matrix, pos_edge_index, neg_edge_index, batch_train_x_index):
  src = jnp.concatenate([pos_edge_index[0], neg_edge_index[0]]).astype(jnp.int32)
  tgt = jnp.concatenate([pos_edge_index[1], neg_edge_index[1]]).astype(jnp.int32)
  return _link_predict(node_embedding_matrix, src, tgt)


# speedup vs baseline: 1.2472x; 1.2472x over previous
import functools

import jax
import jax.numpy as jnp
from jax import lax
from jax.experimental import pallas as pl
from jax.experimental.pallas import tpu as pltpu
from jax.experimental.pallas import tpu_sc as plsc

NC = 2
NS = 16
NW = NC * NS
L = 16

CHUNK = 80
GROUPS = CHUNK // L
IBLK = 25
FLUSH = 10
STRIPE = 1000


def _tec_body(D, per_w, n_nodes, table_hbm, src_hbm, tgt_hbm, out_hbm,
              table_sh, sblk0, tblk0, sblk1, tblk1,
              srows0, trows0, srows1, trows1,
              acc_v, out_v, sem0, sem1, bsem0, bsem1):
  wid = lax.axis_index("s") * NC + lax.axis_index("c")
  sid = lax.axis_index("s")
  n_chunks = per_w // CHUNK
  n_blocks = n_chunks // IBLK
  base = wid * per_w
  nslice = D // L
  gbufs = ((srows0, trows0, sem0), (srows1, trows1, sem1))
  iblks = ((sblk0, tblk0, bsem0), (sblk1, tblk1, bsem1))

  @pl.when(sid < n_nodes // STRIPE)
  def _():
    off = pl.multiple_of(sid * STRIPE, 8)
    pltpu.sync_copy(table_hbm.at[pl.ds(off, STRIPE)], table_sh.at[pl.ds(off, STRIPE)])

  plsc.subcore_barrier()

  def blk_refs(bj):
    off = pl.multiple_of(base + bj * (IBLK * CHUNK), 8)
    return src_hbm.at[pl.ds(off, IBLK * CHUNK)], tgt_hbm.at[pl.ds(off, IBLK * CHUNK)]

  def _for_parity(ci, fn):
    p = lax.rem(ci // IBLK, 2)

    @pl.when(p == 0)
    def _():
      fn(0)

    @pl.when(p == 1)
    def _():
      fn(1)

  def fire_blk(bj):
    shbm, thbm = blk_refs(bj)

    def go(s):
      sblk, tblk, bsem = iblks[s]
      pltpu.async_copy(shbm, sblk, bsem)
      pltpu.async_copy(thbm, tblk, bsem)

    _for_parity(bj * IBLK, go)

  def wait_blk(bj):
    shbm, thbm = blk_refs(bj)

    def go(s):
      sblk, tblk, bsem = iblks[s]
      pltpu.make_async_copy(shbm, sblk, bsem).wait()
      pltpu.make_async_copy(thbm, tblk, bsem).wait()

    _for_parity(bj * IBLK, go)

  def idx_refs(ci, s):
    sblk, tblk, _ = iblks[s]
    off = pl.multiple_of(lax.rem(ci, IBLK) * CHUNK, 8)
    return sblk.at[pl.ds(off, CHUNK)], tblk.at[pl.ds(off, CHUNK)]

  def fire_gather(ci, b):
    srows, trows, sem = gbufs[b]

    def go(s):
      sidx, tidx = idx_refs(ci, s)
      pltpu.async_copy(table_sh.at[sidx], srows, sem)
      pltpu.async_copy(table_sh.at[tidx], trows, sem)

    _for_parity(ci, go)

  def wait_gather(ci, b):
    srows, trows, sem = gbufs[b]

    def go(s):
      sidx, tidx = idx_refs(ci, s)
      pltpu.make_async_copy(table_sh.at[sidx], srows, sem).wait()
      pltpu.make_async_copy(table_sh.at[tidx], trows, sem).wait()

    _for_parity(ci, go)

  def compute(ci, b):
    srows, trows, _ = gbufs[b]
    slot = lax.rem(ci, FLUSH)

    def group_body(g, c2):
      eb = g * L
      for j in range(L):
        e = eb + j
        acc = srows[e, pl.ds(0, L)] * trows[e, pl.ds(0, L)]
        for k in range(1, nslice):
          acc = acc + srows[e, pl.ds(k * L, L)] * trows[e, pl.ds(k * L, L)]
        acc_v[pl.ds(j * L, L)] = acc
      rows = lax.iota(jnp.int32, L) * L
      tot = plsc.load_gather(acc_v, [rows])
      for i in range(1, L):
        tot = tot + plsc.load_gather(acc_v, [rows + i])
      out_v[pl.ds(slot * CHUNK + eb, L)] = 1.0 / (1.0 + jnp.exp(-tot))
      return c2

    lax.fori_loop(0, GROUPS, group_body, 0)

  s0hbm, t0hbm = blk_refs(0)
  pltpu.sync_copy(s0hbm, sblk0)
  pltpu.sync_copy(t0hbm, tblk0)
  fire_blk(1)
  fire_gather(0, 0)

  def outer(io, carry):
    for b in range(2):
      ci = io * 2 + b
      ob = 1 - b

      @pl.when(ci + 1 < n_chunks)
      def _():
        @pl.when(lax.rem(ci + 1, IBLK) == 0)
        def _():
          wait_blk((ci + 1) // IBLK)

        fire_gather(ci + 1, ob)

      wait_gather(ci, b)
      compute(ci, b)

      @pl.when((lax.rem(ci, IBLK) == IBLK - 1) & (ci // IBLK + 2 < n_blocks))
      def _():
        fire_blk(ci // IBLK + 2)

      @pl.when(lax.rem(ci, FLUSH) == FLUSH - 1)
      def _():
        foff = pl.multiple_of(base + (ci - (FLUSH - 1)) * CHUNK, 8)
        pltpu.sync_copy(out_v, out_hbm.at[pl.ds(foff, FLUSH * CHUNK)])

    return carry

  lax.fori_loop(0, n_chunks // 2, outer, 0)


def _link_predict(table, src, tgt):
  E = src.shape[0]
  n_nodes, D = table.shape
  assert E % NW == 0
  per_w = E // NW
  n_chunks = per_w // CHUNK
  assert per_w % CHUNK == 0 and D % L == 0
  assert n_chunks % 2 == 0 and n_chunks % FLUSH == 0 and n_chunks % IBLK == 0
  assert n_nodes % STRIPE == 0 and n_nodes // STRIPE <= NS

  mesh = plsc.VectorSubcoreMesh(core_axis_name="c", subcore_axis_name="s")
  k = pl.kernel(
      functools.partial(_tec_body, D, per_w, n_nodes),
      out_type=jax.ShapeDtypeStruct((E,), jnp.float32),
      mesh=mesh,
      compiler_params=pltpu.CompilerParams(needs_layout_passes=False),
      scratch_types=[
          pltpu.VMEM_SHARED((n_nodes, D), jnp.float32),
          pltpu.VMEM((IBLK * CHUNK,), jnp.int32),
          pltpu.VMEM((IBLK * CHUNK,), jnp.int32),
          pltpu.VMEM((IBLK * CHUNK,), jnp.int32),
          pltpu.VMEM((IBLK * CHUNK,), jnp.int32),
          pltpu.VMEM((CHUNK, D), jnp.float32),
          pltpu.VMEM((CHUNK, D), jnp.float32),
          pltpu.VMEM((CHUNK, D), jnp.float32),
          pltpu.VMEM((CHUNK, D), jnp.float32),
          pltpu.VMEM((L * L,), jnp.float32),
          pltpu.VMEM((FLUSH * CHUNK,), jnp.float32),
          pltpu.SemaphoreType.DMA,
          pltpu.SemaphoreType.DMA,
          pltpu.SemaphoreType.DMA,
          pltpu.SemaphoreType.DMA,
      ],
  )
  return k(table, src, tgt)


def kernel(node_embedding_matrix, pos_edge_index, neg_edge_index, batch_train_x_index):
  src = jnp.concatenate([pos_edge_index[0], neg_edge_index[0]]).astype(jnp.int32)
  tgt = jnp.concatenate([pos_edge_index[1], neg_edge_index[1]]).astype(jnp.int32)
  return _link_predict(node_embedding_matrix, src, tgt)
